# baseline (speedup 1.0000x reference)
import jax
import jax.numpy as jnp
from jax import lax
from jax.experimental import pallas as pl
from jax.experimental.pallas import tpu as pltpu
from jax.experimental.pallas import tpu_sc as plsc

_HIDDEN = 64
_B = 4096 * 200
_NC = 2
_NW = 32
_BPW = _B // _NW
_CHUNK = 512
_NCHUNK = _BPW // _CHUNK
_SCALE = 8.0

_mesh = plsc.VectorSubcoreMesh(core_axis_name="c", subcore_axis_name="s")


def _body(table_hbm, idx_hbm, out_hbm, idx_v, rows_v, gsem):
    wid = lax.axis_index("s") * _NC + lax.axis_index("c")
    base = wid * _BPW
    pltpu.sync_copy(idx_hbm.at[pl.ds(base, _BPW)], idx_v)

    @pl.loop(0, _NCHUNK)
    def _chunk(g):
        row0 = g * _CHUNK
        pltpu.async_copy(
            table_hbm.at[idx_v.at[pl.ds(row0, _CHUNK)]], rows_v, gsem
        ).wait()

        @pl.loop(0, _CHUNK)
        def _scale_row(r):
            for c in range(_HIDDEN // 16):
                sl = pl.ds(c * 16, 16)
                rows_v[r, sl] = rows_v[r, sl] * _SCALE

        pltpu.sync_copy(rows_v, out_hbm.at[pl.ds(base + row0, _CHUNK)])


_lookup = pl.kernel(
    _body,
    out_type=jax.ShapeDtypeStruct((_B, _HIDDEN), jnp.float32),
    mesh=_mesh,
    scratch_types=[
        pltpu.VMEM((_BPW,), jnp.int32),
        pltpu.VMEM((_CHUNK, _HIDDEN), jnp.float32),
        pltpu.SemaphoreType.DMA,
    ],
    compiler_params=pltpu.CompilerParams(use_tc_tiling_on_sc=False),
)


def kernel(input_ids, table):
    idx = input_ids.reshape(-1).astype(jnp.int32)
    out = _lookup(table, idx)
    return out.reshape(*input_ids.shape, _HIDDEN)


# speedup vs baseline: 1.1165x; 1.1165x over previous
import jax
import jax.numpy as jnp
from jax import lax
from jax.experimental import pallas as pl
from jax.experimental.pallas import tpu as pltpu
from jax.experimental.pallas import tpu_sc as plsc

_HIDDEN = 64
_B = 4096 * 200
_NC = 2
_NW = 32
_BPW = _B // _NW
_CHUNK = 256
_NCHUNK = _BPW // _CHUNK
_NBUF = 4
_SCALE = 8.0

_mesh = plsc.VectorSubcoreMesh(core_axis_name="c", subcore_axis_name="s")


def _body(table_hbm, idx_hbm, out_hbm, idx_v, bufs, gsems, wsems):
    wid = lax.axis_index("s") * _NC + lax.axis_index("c")
    base = wid * _BPW
    pltpu.sync_copy(idx_hbm.at[pl.ds(base, _BPW)], idx_v)

    def start_gather(c, b):
        pltpu.async_copy(
            table_hbm.at[idx_v.at[pl.ds(c * _CHUNK, _CHUNK)]], bufs[b], gsems[b])

    def wait_gather(b):
        pltpu.make_async_copy(
            table_hbm.at[pl.ds(0, _CHUNK)], bufs[b], gsems[b]).wait()

    def start_writeback(c, b):
        pltpu.async_copy(
            bufs[b], out_hbm.at[pl.ds(base + c * _CHUNK, _CHUNK)], wsems[b])

    def wait_writeback(b):
        pltpu.make_async_copy(
            bufs[b], out_hbm.at[pl.ds(base, _CHUNK)], wsems[b]).wait()

    start_gather(0, 0)
    start_gather(1, 1)

    @pl.loop(0, _NCHUNK, step=_NBUF)
    def _grp(g):
        for b in range(_NBUF):
            c = g + b
            bp = (b + 2) % _NBUF

            @pl.when(c >= 2)
            def _():
                wait_writeback(bp)

            @pl.when(c + 2 < _NCHUNK)
            def _():
                start_gather(c + 2, bp)

            wait_gather(b)

            buf = bufs[b]

            @pl.loop(0, _CHUNK, unroll=4)
            def _scale_row(r):
                for col in range(_HIDDEN // 16):
                    sl = pl.ds(col * 16, 16)
                    buf[r, sl] = buf[r, sl] * _SCALE

            start_writeback(c, b)

    wait_writeback((_NCHUNK - 2) % _NBUF)
    wait_writeback((_NCHUNK - 1) % _NBUF)


_lookup = pl.kernel(
    _body,
    out_type=jax.ShapeDtypeStruct((_B, _HIDDEN), jnp.float32),
    mesh=_mesh,
    scratch_types=[
        pltpu.VMEM((_BPW,), jnp.int32),
        [pltpu.VMEM((_CHUNK, _HIDDEN), jnp.float32) for _ in range(_NBUF)],
        [pltpu.SemaphoreType.DMA for _ in range(_NBUF)],
        [pltpu.SemaphoreType.DMA for _ in range(_NBUF)],
    ],
    compiler_params=pltpu.CompilerParams(use_tc_tiling_on_sc=False),
)


def kernel(input_ids, table):
    idx = input_ids.reshape(-1).astype(jnp.int32)
    out = _lookup(table, idx)
    return out.reshape(*input_ids.shape, _HIDDEN)


# speedup vs baseline: 1.1175x; 1.0008x over previous
import jax
import jax.numpy as jnp
from jax import lax
from jax.experimental import pallas as pl
from jax.experimental.pallas import tpu as pltpu
from jax.experimental.pallas import tpu_sc as plsc

_HIDDEN = 64
_ROWS = 4096
_COLS = 200
_NC = 2
_NW = 32
_RPW = _ROWS // _NW
_NBUF = 4
_SCALE = 8.0

_mesh = plsc.VectorSubcoreMesh(core_axis_name="c", subcore_axis_name="s")


def _body(table_hbm, idx_hbm, out_hbm, idx_v, bufs, gsems, wsems):
    wid = lax.axis_index("s") * _NC + lax.axis_index("c")
    row0 = wid * _RPW
    pltpu.sync_copy(idx_hbm.at[pl.ds(row0, _RPW)], idx_v)

    def start_gather(r, b):
        pltpu.async_copy(table_hbm.at[idx_v.at[r]], bufs[b], gsems[b])

    def wait_gather(b):
        pltpu.make_async_copy(
            table_hbm.at[pl.ds(0, _COLS)], bufs[b], gsems[b]).wait()

    def start_writeback(r, b):
        pltpu.async_copy(bufs[b], out_hbm.at[row0 + r], wsems[b])

    def wait_writeback(b):
        pltpu.make_async_copy(bufs[b], out_hbm.at[row0], wsems[b]).wait()

    start_gather(0, 0)
    start_gather(1, 1)

    @pl.loop(0, _RPW, step=_NBUF)
    def _grp(g):
        for b in range(_NBUF):
            r = g + b
            bp = (b + 2) % _NBUF

            @pl.when(r >= 2)
            def _():
                wait_writeback(bp)

            @pl.when(r + 2 < _RPW)
            def _():
                start_gather(r + 2, bp)

            wait_gather(b)

            buf = bufs[b]

            @pl.loop(0, _COLS, unroll=4)
            def _scale_row(j):
                for col in range(_HIDDEN // 16):
                    sl = pl.ds(col * 16, 16)
                    buf[j, sl] = buf[j, sl] * _SCALE

            start_writeback(r, b)

    wait_writeback((_RPW - 2) % _NBUF)
    wait_writeback((_RPW - 1) % _NBUF)


_lookup = pl.kernel(
    _body,
    out_type=jax.ShapeDtypeStruct((_ROWS, _COLS, _HIDDEN), jnp.float32),
    mesh=_mesh,
    scratch_types=[
        pltpu.VMEM((_RPW, _COLS), jnp.int32),
        [pltpu.VMEM((_COLS, _HIDDEN), jnp.float32) for _ in range(_NBUF)],
        [pltpu.SemaphoreType.DMA for _ in range(_NBUF)],
        [pltpu.SemaphoreType.DMA for _ in range(_NBUF)],
    ],
    compiler_params=pltpu.CompilerParams(use_tc_tiling_on_sc=False),
)


def kernel(input_ids, table):
    return _lookup(table, input_ids.astype(jnp.int32))


# speedup vs baseline: 1.6630x; 1.4882x over previous
import jax
import jax.numpy as jnp
from jax import lax
from jax.experimental import pallas as pl
from jax.experimental.pallas import tpu as pltpu
from jax.experimental.pallas import tpu_sc as plsc

_HIDDEN = 64
_B = 4096 * 200
_NC = 2
_NW = 32
_BPW = _B // _NW
_CHUNK = 128
_NCHUNK = _BPW // _CHUNK
_NBUF = 4
_SCALE = 8.0

_mesh = plsc.VectorSubcoreMesh(core_axis_name="c", subcore_axis_name="s")


def _body(table_hbm, idx_hbm, out_hbm, idx_v, bufs, gsems, wsems):
    wid = lax.axis_index("s") * _NC + lax.axis_index("c")
    base = wid * _BPW
    pltpu.sync_copy(idx_hbm.at[pl.ds(base, _BPW)], idx_v)

    def start_gather(c, b):
        buf, sem = bufs[b], gsems[b]

        @pl.loop(0, _CHUNK // 16)
        def _grp16(k):
            v = idx_v[pl.ds(c * _CHUNK + k * 16, 16)]
            for l in range(16):
                pltpu.async_copy(
                    table_hbm.at[v[l]], buf.at[k * 16 + l], sem)

    def wait_gather(b):
        pltpu.make_async_copy(
            table_hbm.at[pl.ds(0, _CHUNK)], bufs[b], gsems[b]).wait()

    def start_writeback(c, b):
        pltpu.async_copy(
            bufs[b], out_hbm.at[pl.ds(base + c * _CHUNK, _CHUNK)], wsems[b])

    def wait_writeback(b):
        pltpu.make_async_copy(
            bufs[b], out_hbm.at[pl.ds(base, _CHUNK)], wsems[b]).wait()

    start_gather(0, 0)
    start_gather(1, 1)

    @pl.loop(0, _NCHUNK, step=_NBUF)
    def _grp(g):
        for b in range(_NBUF):
            c = g + b
            bp = (b + 2) % _NBUF

            @pl.when(c >= 2)
            def _():
                wait_writeback(bp)

            @pl.when(c + 2 < _NCHUNK)
            def _():
                start_gather(c + 2, bp)

            wait_gather(b)

            buf = bufs[b]

            @pl.loop(0, _CHUNK, unroll=4)
            def _scale_row(j):
                for col in range(_HIDDEN // 16):
                    sl = pl.ds(col * 16, 16)
                    buf[j, sl] = buf[j, sl] * _SCALE

            start_writeback(c, b)

    wait_writeback((_NCHUNK - 2) % _NBUF)
    wait_writeback((_NCHUNK - 1) % _NBUF)


_lookup = pl.kernel(
    _body,
    out_type=jax.ShapeDtypeStruct((_B, _HIDDEN), jnp.float32),
    mesh=_mesh,
    scratch_types=[
        pltpu.VMEM((_BPW,), jnp.int32),
        [pltpu.VMEM((_CHUNK, _HIDDEN), jnp.float32) for _ in range(_NBUF)],
        [pltpu.SemaphoreType.DMA for _ in range(_NBUF)],
        [pltpu.SemaphoreType.DMA for _ in range(_NBUF)],
    ],
    compiler_params=pltpu.CompilerParams(use_tc_tiling_on_sc=True),
)


def kernel(input_ids, table):
    idx = input_ids.reshape(-1).astype(jnp.int32)
    out = _lookup(table, idx)
    return out.reshape(*input_ids.shape, _HIDDEN)


# speedup vs baseline: 1.6696x; 1.0039x over previous
import jax
import jax.numpy as jnp
from jax import lax
from jax.experimental import pallas as pl
from jax.experimental.pallas import tpu as pltpu
from jax.experimental.pallas import tpu_sc as plsc

_HIDDEN = 64
_B = 4096 * 200
_NC = 2
_NW = 32
_BPW = _B // _NW
_CHUNK = 128
_NCHUNK = _BPW // _CHUNK
_NBUF = 4
_SCALE = 8.0

_mesh = plsc.VectorSubcoreMesh(core_axis_name="c", subcore_axis_name="s")


def _body(table_hbm, idx_hbm, out_hbm, idx_v, bufs, gsems, wsems):
    wid = lax.axis_index("s") * _NC + lax.axis_index("c")
    base = wid * _BPW
    pltpu.sync_copy(idx_hbm.at[pl.ds(base, _BPW)], idx_v)

    def start_gather(c, b):
        buf, sem = bufs[b], gsems[b]

        @pl.loop(0, _CHUNK // 16, unroll=4)
        def _grp16(k):
            v = idx_v[pl.ds(c * _CHUNK + k * 16, 16)]
            for l in range(16):
                pltpu.async_copy(
                    table_hbm.at[v[l]], buf.at[k * 16 + l], sem)

    def wait_gather(b):
        pltpu.make_async_copy(
            table_hbm.at[pl.ds(0, _CHUNK)], bufs[b], gsems[b]).wait()

    def start_writeback(c, b):
        pltpu.async_copy(
            bufs[b], out_hbm.at[pl.ds(base + c * _CHUNK, _CHUNK)], wsems[b])

    def wait_writeback(b):
        pltpu.make_async_copy(
            bufs[b], out_hbm.at[pl.ds(base, _CHUNK)], wsems[b]).wait()

    start_gather(0, 0)
    start_gather(1, 1)

    @pl.loop(0, _NCHUNK, step=_NBUF)
    def _grp(g):
        for b in range(_NBUF):
            c = g + b
            bp = (b + 2) % _NBUF

            @pl.when(c >= 2)
            def _():
                wait_writeback(bp)

            @pl.when(c + 2 < _NCHUNK)
            def _():
                start_gather(c + 2, bp)

            wait_gather(b)

            buf = bufs[b]

            @pl.loop(0, _CHUNK, unroll=4)
            def _scale_row(j):
                for col in range(_HIDDEN // 16):
                    sl = pl.ds(col * 16, 16)
                    buf[j, sl] = buf[j, sl] * _SCALE

            start_writeback(c, b)

    wait_writeback((_NCHUNK - 2) % _NBUF)
    wait_writeback((_NCHUNK - 1) % _NBUF)


_lookup = pl.kernel(
    _body,
    out_type=jax.ShapeDtypeStruct((_B, _HIDDEN), jnp.float32),
    mesh=_mesh,
    scratch_types=[
        pltpu.VMEM((_BPW,), jnp.int32),
        [pltpu.VMEM((_CHUNK, _HIDDEN), jnp.float32) for _ in range(_NBUF)],
        [pltpu.SemaphoreType.DMA for _ in range(_NBUF)],
        [pltpu.SemaphoreType.DMA for _ in range(_NBUF)],
    ],
    compiler_params=pltpu.CompilerParams(use_tc_tiling_on_sc=True),
)


def kernel(input_ids, table):
    idx = input_ids.reshape(-1).astype(jnp.int32)
    out = _lookup(table, idx)
    return out.reshape(*input_ids.shape, _HIDDEN)
